# baseline (speedup 1.0000x reference)
import functools

import jax
import jax.numpy as jnp
from jax import lax
from jax.experimental import pallas as pl
from jax.experimental.pallas import tpu as pltpu
from jax.experimental.pallas import tpu_sc as plsc

NNEIGH = 16
TBLW = 256
XYZW = 16
TQ_TOPK = 256
TQ_ATT = 128
SC_CORES = 2
SC_SUBCORES = 16
SC_CHUNK = 128


def _prep_body(points_ref, lat_ref, W_qs_ref, W_ks_ref, W_vs_ref,
               W_kg_ref, W_vg_ref, d_b2_ref, g_w1_ref, g_b1_ref, g_w2_ref,
               g_b2_ref, d_w2_ref, table_ref, cvec_ref, lgl_ref, vg_ref,
               wdg_ref, dw2b_ref, gw2b_ref):
    pts = points_ref[0]
    g_w1 = g_w1_ref[...]
    W_ksg = jnp.dot(W_ks_ref[...], g_w1, preferred_element_type=jnp.float32)
    Kg = jnp.dot(pts, W_ksg, preferred_element_type=jnp.float32)
    V = jnp.dot(pts, W_vs_ref[...], preferred_element_type=jnp.float32)
    kg_b = lax.bitcast_convert_type(Kg.astype(jnp.bfloat16), jnp.int16)
    v_b = lax.bitcast_convert_type(V.astype(jnp.bfloat16), jnp.int16)
    table_ref[...] = ((kg_b.astype(jnp.int32) & 0xFFFF)
                      | (v_b.astype(jnp.int32) << 16))

    lat = lat_ref[...]
    q = jnp.dot(lat, W_qs_ref[...], preferred_element_type=jnp.float32)
    qg = jnp.dot(q, g_w1, preferred_element_type=jnp.float32)
    b2g = jnp.dot(d_b2_ref[...], g_w1, preferred_element_type=jnp.float32)
    cvec_ref[...] = qg + b2g + g_b1_ref[...]
    kglob = jnp.dot(lat, W_kg_ref[...], preferred_element_type=jnp.float32)
    hg = jnp.maximum(
        jnp.dot(q - kglob, g_w1, preferred_element_type=jnp.float32)
        + g_b1_ref[...], 0.0)
    lgl_ref[...] = (jnp.dot(hg, g_w2_ref[...],
                            preferred_element_type=jnp.float32)
                    + g_b2_ref[...])
    vg_ref[...] = jnp.dot(lat, W_vg_ref[...],
                          preferred_element_type=jnp.float32)
    wdg = jnp.dot(d_w2_ref[...], g_w1, preferred_element_type=jnp.float32)
    wdg_ref[...] = wdg.astype(jnp.bfloat16)
    dw2b_ref[...] = d_w2_ref[...].astype(jnp.bfloat16)
    gw2b_ref[...] = g_w2_ref[...].astype(jnp.bfloat16)


def _prep_call(points, lat_rep, W_qs, W_ks, W_vs, W_kg, W_vg, d_b2,
               g_w1, g_b1, g_w2, g_b2, d_w2):
    B, N, DI = points.shape
    DIM = W_qs.shape[1]
    full = lambda s: pl.BlockSpec(s, lambda b: tuple(0 for _ in s))
    return pl.pallas_call(
        _prep_body,
        grid=(B,),
        in_specs=[
            pl.BlockSpec((1, N, DI), lambda b: (b, 0, 0)),
            full((B, DI)), full((DI, DIM)), full((DI, DIM)), full((DI, DIM)),
            full((DI, DIM)), full((DI, DIM)), full((1, DIM)),
            full((DIM, DIM)), full((1, DIM)), full((DIM, DIM)),
            full((1, DIM)), full((DIM, DIM)),
        ],
        out_specs=[
            pl.BlockSpec((N, TBLW), lambda b: (b, 0)),
            full((B, DIM)), full((B, DIM)), full((B, DIM)),
            full((DIM, DIM)), full((DIM, DIM)), full((DIM, DIM)),
        ],
        out_shape=[
            jax.ShapeDtypeStruct((B * N, TBLW), jnp.int32),
            jax.ShapeDtypeStruct((B, DIM), jnp.float32),
            jax.ShapeDtypeStruct((B, DIM), jnp.float32),
            jax.ShapeDtypeStruct((B, DIM), jnp.float32),
            jax.ShapeDtypeStruct((DIM, DIM), jnp.bfloat16),
            jax.ShapeDtypeStruct((DIM, DIM), jnp.bfloat16),
            jax.ShapeDtypeStruct((DIM, DIM), jnp.bfloat16),
        ],
    )(points, lat_rep, W_qs, W_ks, W_vs, W_kg, W_vg, d_b2, g_w1, g_b1,
      g_w2, g_b2, d_w2)


def _topk_body(xq_ref, xzT_ref, out_ref, d_ref, *, n):
    b = pl.program_id(0)
    xq = xq_ref[0]
    xzT = xzT_ref[0]
    xn = jnp.sum(xzT * xzT, axis=0, keepdims=True)
    qn = jnp.sum(xq * xq, axis=1, keepdims=True)
    cross = lax.dot_general(xq, xzT, (((1,), (0,)), ((), ())),
                            preferred_element_type=jnp.float32)
    d0 = (qn + xn) - 2.0 * cross
    d_ref[...] = d0
    tq = d0.shape[0]
    cq = 256

    def chunk(c, carry):
        dd = d_ref[pl.ds(c * cq, cq), :]
        iota = lax.broadcasted_iota(jnp.int32, (cq, n), 1)
        lane_k = lax.broadcasted_iota(jnp.int32, (cq, NNEIGH), 1)

        def step(k, kc):
            d, acc = kc
            m = jnp.min(d, axis=1, keepdims=True)
            idx = jnp.min(jnp.where(d <= m, iota, n), axis=1, keepdims=True)
            acc = jnp.where(lane_k == k, idx, acc)
            d = jnp.where(iota == idx, jnp.float32(3e38), d)
            return d, acc

        _, acc = lax.fori_loop(
            0, NNEIGH, step, (dd, jnp.zeros((cq, NNEIGH), jnp.int32)))
        out_ref[0, pl.ds(c * cq, cq), :] = acc + b * n
        return carry

    lax.fori_loop(0, tq // cq, chunk, 0)


def _topk_call(xyz_q, xyzT):
    B, NQ, _ = xyz_q.shape
    N = xyzT.shape[2]
    nt = NQ // TQ_TOPK
    return pl.pallas_call(
        functools.partial(_topk_body, n=N),
        grid=(B, nt),
        in_specs=[
            pl.BlockSpec((1, TQ_TOPK, 3), lambda b, t: (b, t, 0)),
            pl.BlockSpec((1, 3, N), lambda b, t: (b, 0, 0)),
        ],
        out_specs=pl.BlockSpec((1, TQ_TOPK, NNEIGH), lambda b, t: (b, t, 0)),
        out_shape=jax.ShapeDtypeStruct((B, NQ, NNEIGH), jnp.int32),
        scratch_shapes=[pltpu.VMEM((TQ_TOPK, N), jnp.float32)],
    )(xyz_q, xyzT)


def _gather_body(table_hbm, gidx_hbm, out_hbm, idx_v, rows_v, sem, *,
                 rows_per_worker):
    wid = (lax.axis_index("s") * SC_CORES + lax.axis_index("c"))
    base = wid * rows_per_worker
    pltpu.sync_copy(gidx_hbm.at[pl.ds(base, rows_per_worker)], idx_v)

    def chunk(i, carry):
        off = pl.multiple_of(base + i * SC_CHUNK, 8)
        loff = pl.multiple_of(i * SC_CHUNK, 8)
        idx = idx_v.at[pl.ds(loff, SC_CHUNK)]
        pltpu.async_copy(table_hbm.at[idx], rows_v, sem).wait()
        pltpu.sync_copy(rows_v, out_hbm.at[pl.ds(off, SC_CHUNK)])
        return carry

    lax.fori_loop(0, rows_per_worker // SC_CHUNK, chunk, 0)


def _gather_call(table, gidx_flat, width, dtype, tc_tiling):
    rows = gidx_flat.shape[0]
    nw = SC_CORES * SC_SUBCORES
    rpw = rows // nw
    mesh = plsc.VectorSubcoreMesh(core_axis_name="c", subcore_axis_name="s",
                                  num_cores=SC_CORES,
                                  num_subcores=SC_SUBCORES)
    return pl.kernel(
        functools.partial(_gather_body, rows_per_worker=rpw),
        out_type=jax.ShapeDtypeStruct((rows, width), dtype),
        mesh=mesh,
        compiler_params=pltpu.CompilerParams(use_tc_tiling_on_sc=tc_tiling),
        scratch_types=[
            pltpu.VMEM((rpw,), jnp.int32),
            pltpu.VMEM((SC_CHUNK, width), dtype),
            pltpu.SemaphoreType.DMA,
        ],
    )(table, gidx_flat)


def _attend_body(g_ref, xg_ref, xq_ref, cvec_ref, lgl_ref, vg_ref, wdg_ref,
                 d_w1_ref, d_b1_ref, d_w2_ref, d_b2_ref, g_w2_ref, g_b2_ref,
                 out_ref):
    G = g_ref[...]
    tq = out_ref.shape[1]
    k = NNEIGH
    dim = out_ref.shape[2]
    Kg = lax.bitcast_convert_type(
        (G & 0xFFFF).astype(jnp.int16), jnp.bfloat16).astype(jnp.float32)
    V = lax.bitcast_convert_type(
        (G >> 16).astype(jnp.int16), jnp.bfloat16).astype(jnp.float32)
    xg = xg_ref[...][:, :3]
    xq = xq_ref[0]
    xqr = jnp.broadcast_to(xq[:, None, :], (tq, k, 3)).reshape(tq * k, 3)
    rel = xqr - xg
    hd = jnp.maximum(
        jnp.dot(rel, d_w1_ref[...], preferred_element_type=jnp.float32)
        + d_b1_ref[...], 0.0)
    hdb = hd.astype(jnp.bfloat16)
    posg = jnp.dot(hdb, wdg_ref[...], preferred_element_type=jnp.float32)
    cvec = cvec_ref[0]
    hidden = jnp.maximum(cvec - Kg + posg, 0.0).astype(jnp.bfloat16)
    logits = (jnp.dot(hidden, g_w2_ref[...],
                      preferred_element_type=jnp.float32) + g_b2_ref[...])
    pos2 = (jnp.dot(hdb, d_w2_ref[...], preferred_element_type=jnp.float32)
            + d_b2_ref[...])
    w = V + pos2

    L3 = logits.reshape(tq, k, dim)
    W3 = w.reshape(tq, k, dim)
    lg = lgl_ref[0]
    m = jnp.maximum(jnp.max(L3, axis=1), lg)
    e = jnp.exp(L3 - m[:, None, :])
    eg = jnp.exp(lg - m)
    denom = jnp.sum(e, axis=1) + eg
    num = jnp.sum(e * W3, axis=1) + eg * vg_ref[0]
    out_ref[0] = num / denom


def _attend_call(g2, gx2, xyz_q, cvec, lgl, vg, wdg, d_w1, d_b1, d_w2, d_b2,
                 g_w2, g_b2):
    B, NQ, _ = xyz_q.shape
    DIM = wdg.shape[0]
    nt = NQ // TQ_ATT
    R = TQ_ATT * NNEIGH
    full = lambda s: pl.BlockSpec(s, lambda b, t: tuple(0 for _ in s))
    return pl.pallas_call(
        _attend_body,
        grid=(B, nt),
        in_specs=[
            pl.BlockSpec((R, TBLW), lambda b, t: (b * nt + t, 0)),
            pl.BlockSpec((R, XYZW), lambda b, t: (b * nt + t, 0)),
            pl.BlockSpec((1, TQ_ATT, 3), lambda b, t: (b, t, 0)),
            pl.BlockSpec((1, 1, DIM), lambda b, t: (b, 0, 0)),
            pl.BlockSpec((1, 1, DIM), lambda b, t: (b, 0, 0)),
            pl.BlockSpec((1, 1, DIM), lambda b, t: (b, 0, 0)),
            full((DIM, DIM)), full((3, DIM)), full((1, DIM)),
            full((DIM, DIM)), full((1, DIM)), full((DIM, DIM)),
            full((1, DIM)),
        ],
        out_specs=pl.BlockSpec((1, TQ_ATT, DIM), lambda b, t: (b, t, 0)),
        out_shape=jax.ShapeDtypeStruct((B, NQ, DIM), jnp.float32),
    )(g2, gx2, xyz_q, cvec, lgl, vg, wdg, d_w1, d_b1, d_w2, d_b2, g_w2, g_b2)


def kernel(xyz_q, lat_rep, xyz, points, W_qs, W_ks, W_vs, W_kg, W_vg,
           d_w1, d_b1, d_w2, d_b2, g_w1, g_b1, g_w2, g_b2):
    B, NQ, _ = xyz_q.shape
    N = xyz.shape[1]
    DIM = W_qs.shape[1]
    r2 = lambda v: v.reshape(1, DIM)

    table, cvec, lgl, vg, wdg, d_w2b, g_w2b = _prep_call(
        points, lat_rep, W_qs, W_ks, W_vs, W_kg, W_vg, r2(d_b2),
        g_w1, r2(g_b1), g_w2, r2(g_b2), d_w2)
    gidx = _topk_call(xyz_q, xyz.transpose(0, 2, 1)).reshape(-1)
    xyzt = jnp.pad(xyz, ((0, 0), (0, 0), (0, XYZW - 3))).reshape(B * N, XYZW)
    g = _gather_call(table, gidx, TBLW, jnp.int32, True)
    gx = _gather_call(xyzt, gidx, XYZW, jnp.float32, False)
    r3 = lambda a: a.reshape(B, 1, DIM)
    return _attend_call(g, gx, xyz_q, r3(cvec), r3(lgl), r3(vg), wdg,
                        d_w1, r2(d_b1), d_w2b, r2(d_b2), g_w2b, r2(g_b2))


# speedup vs baseline: 1.0552x; 1.0552x over previous
import functools

import jax
import jax.numpy as jnp
from jax import lax
from jax.experimental import pallas as pl
from jax.experimental.pallas import tpu as pltpu
from jax.experimental.pallas import tpu_sc as plsc

NNEIGH = 16
TBLW = 256
XYZW = 16
TQ_TOPK = 256
TQ_ATT = 128
SC_CORES = 2
SC_SUBCORES = 16
SC_CHUNK = 128


def _prep_body(points_ref, lat_ref, W_qs_ref, W_ks_ref, W_vs_ref,
               W_kg_ref, W_vg_ref, d_b2_ref, g_w1_ref, g_b1_ref, g_w2_ref,
               g_b2_ref, d_w2_ref, table_ref, cvec_ref, lgl_ref, vg_ref,
               wdg_ref, dw2b_ref, gw2b_ref):
    pts = points_ref[0]
    g_w1 = g_w1_ref[...]
    W_ksg = jnp.dot(W_ks_ref[...], g_w1, preferred_element_type=jnp.float32)
    Kg = jnp.dot(pts, W_ksg, preferred_element_type=jnp.float32)
    V = jnp.dot(pts, W_vs_ref[...], preferred_element_type=jnp.float32)
    kg_b = lax.bitcast_convert_type(Kg.astype(jnp.bfloat16), jnp.int16)
    v_b = lax.bitcast_convert_type(V.astype(jnp.bfloat16), jnp.int16)
    table_ref[...] = ((kg_b.astype(jnp.int32) & 0xFFFF)
                      | (v_b.astype(jnp.int32) << 16))

    lat = lat_ref[...]
    q = jnp.dot(lat, W_qs_ref[...], preferred_element_type=jnp.float32)
    qg = jnp.dot(q, g_w1, preferred_element_type=jnp.float32)
    b2g = jnp.dot(d_b2_ref[...], g_w1, preferred_element_type=jnp.float32)
    cvec_ref[...] = qg + b2g + g_b1_ref[...]
    kglob = jnp.dot(lat, W_kg_ref[...], preferred_element_type=jnp.float32)
    hg = jnp.maximum(
        jnp.dot(q - kglob, g_w1, preferred_element_type=jnp.float32)
        + g_b1_ref[...], 0.0)
    lgl_ref[...] = (jnp.dot(hg, g_w2_ref[...],
                            preferred_element_type=jnp.float32)
                    + g_b2_ref[...])
    vg_ref[...] = jnp.dot(lat, W_vg_ref[...],
                          preferred_element_type=jnp.float32)
    wdg = jnp.dot(d_w2_ref[...], g_w1, preferred_element_type=jnp.float32)
    wdg_ref[...] = wdg.astype(jnp.bfloat16)
    dw2b_ref[...] = d_w2_ref[...].astype(jnp.bfloat16)
    gw2b_ref[...] = g_w2_ref[...].astype(jnp.bfloat16)


def _prep_call(points, lat_rep, W_qs, W_ks, W_vs, W_kg, W_vg, d_b2,
               g_w1, g_b1, g_w2, g_b2, d_w2):
    B, N, DI = points.shape
    DIM = W_qs.shape[1]
    full = lambda s: pl.BlockSpec(s, lambda b: tuple(0 for _ in s))
    return pl.pallas_call(
        _prep_body,
        grid=(B,),
        in_specs=[
            pl.BlockSpec((1, N, DI), lambda b: (b, 0, 0)),
            full((B, DI)), full((DI, DIM)), full((DI, DIM)), full((DI, DIM)),
            full((DI, DIM)), full((DI, DIM)), full((1, DIM)),
            full((DIM, DIM)), full((1, DIM)), full((DIM, DIM)),
            full((1, DIM)), full((DIM, DIM)),
        ],
        out_specs=[
            pl.BlockSpec((N, TBLW), lambda b: (b, 0)),
            full((B, DIM)), full((B, DIM)), full((B, DIM)),
            full((DIM, DIM)), full((DIM, DIM)), full((DIM, DIM)),
        ],
        out_shape=[
            jax.ShapeDtypeStruct((B * N, TBLW), jnp.int32),
            jax.ShapeDtypeStruct((B, DIM), jnp.float32),
            jax.ShapeDtypeStruct((B, DIM), jnp.float32),
            jax.ShapeDtypeStruct((B, DIM), jnp.float32),
            jax.ShapeDtypeStruct((DIM, DIM), jnp.bfloat16),
            jax.ShapeDtypeStruct((DIM, DIM), jnp.bfloat16),
            jax.ShapeDtypeStruct((DIM, DIM), jnp.bfloat16),
        ],
    )(points, lat_rep, W_qs, W_ks, W_vs, W_kg, W_vg, d_b2, g_w1, g_b1,
      g_w2, g_b2, d_w2)


def _topk_body(xq_ref, xzT_ref, out_ref, d_ref, *, n, boff):
    b = pl.program_id(0) + boff
    xq = xq_ref[0]
    xzT = xzT_ref[0]
    xn = jnp.sum(xzT * xzT, axis=0, keepdims=True)
    qn = jnp.sum(xq * xq, axis=1, keepdims=True)
    cross = lax.dot_general(xq, xzT, (((1,), (0,)), ((), ())),
                            preferred_element_type=jnp.float32)
    d0 = (qn + xn) - 2.0 * cross
    d_ref[...] = d0
    tq = d0.shape[0]
    cq = 256

    def chunk(c, carry):
        dd = d_ref[pl.ds(c * cq, cq), :]
        iota = lax.broadcasted_iota(jnp.int32, (cq, n), 1)
        lane_k = lax.broadcasted_iota(jnp.int32, (cq, NNEIGH), 1)

        def step(k, kc):
            d, acc = kc
            m = jnp.min(d, axis=1, keepdims=True)
            idx = jnp.min(jnp.where(d <= m, iota, n), axis=1, keepdims=True)
            acc = jnp.where(lane_k == k, idx, acc)
            d = jnp.where(iota == idx, jnp.float32(3e38), d)
            return d, acc

        _, acc = lax.fori_loop(
            0, NNEIGH, step, (dd, jnp.zeros((cq, NNEIGH), jnp.int32)))
        out_ref[0, pl.ds(c * cq, cq), :] = acc + b * n
        return carry

    lax.fori_loop(0, tq // cq, chunk, 0)


def _topk_call(xyz_q, xyzT, boff=0):
    B, NQ, _ = xyz_q.shape
    N = xyzT.shape[2]
    nt = NQ // TQ_TOPK
    return pl.pallas_call(
        functools.partial(_topk_body, n=N, boff=boff),
        grid=(B, nt),
        in_specs=[
            pl.BlockSpec((1, TQ_TOPK, 3), lambda b, t: (b, t, 0)),
            pl.BlockSpec((1, 3, N), lambda b, t: (b, 0, 0)),
        ],
        out_specs=pl.BlockSpec((1, TQ_TOPK, NNEIGH), lambda b, t: (b, t, 0)),
        out_shape=jax.ShapeDtypeStruct((B, NQ, NNEIGH), jnp.int32),
        scratch_shapes=[pltpu.VMEM((TQ_TOPK, N), jnp.float32)],
    )(xyz_q, xyzT)


def _gather_body(table_hbm, gidx_hbm, out_hbm, idx_v, rows_v, sem, *,
                 rows_per_worker):
    wid = (lax.axis_index("s") * SC_CORES + lax.axis_index("c"))
    base = wid * rows_per_worker
    pltpu.sync_copy(gidx_hbm.at[pl.ds(base, rows_per_worker)], idx_v)

    def chunk(i, carry):
        off = pl.multiple_of(base + i * SC_CHUNK, 8)
        loff = pl.multiple_of(i * SC_CHUNK, 8)
        idx = idx_v.at[pl.ds(loff, SC_CHUNK)]
        pltpu.async_copy(table_hbm.at[idx], rows_v, sem).wait()
        pltpu.sync_copy(rows_v, out_hbm.at[pl.ds(off, SC_CHUNK)])
        return carry

    lax.fori_loop(0, rows_per_worker // SC_CHUNK, chunk, 0)


def _gather_call(table, gidx_flat, width, dtype, tc_tiling):
    rows = gidx_flat.shape[0]
    nw = SC_CORES * SC_SUBCORES
    rpw = rows // nw
    mesh = plsc.VectorSubcoreMesh(core_axis_name="c", subcore_axis_name="s",
                                  num_cores=SC_CORES,
                                  num_subcores=SC_SUBCORES)
    return pl.kernel(
        functools.partial(_gather_body, rows_per_worker=rpw),
        out_type=jax.ShapeDtypeStruct((rows, width), dtype),
        mesh=mesh,
        compiler_params=pltpu.CompilerParams(use_tc_tiling_on_sc=tc_tiling),
        scratch_types=[
            pltpu.VMEM((rpw,), jnp.int32),
            pltpu.VMEM((SC_CHUNK, width), dtype),
            pltpu.SemaphoreType.DMA,
        ],
    )(table, gidx_flat)


def _attend_body(g_ref, xg_ref, xq_ref, cvec_ref, lgl_ref, vg_ref, wdg_ref,
                 d_w1_ref, d_b1_ref, d_w2_ref, d_b2_ref, g_w2_ref, g_b2_ref,
                 out_ref):
    G = g_ref[...]
    tq = out_ref.shape[1]
    k = NNEIGH
    dim = out_ref.shape[2]
    Kg = lax.bitcast_convert_type(
        (G & 0xFFFF).astype(jnp.int16), jnp.bfloat16).astype(jnp.float32)
    V = lax.bitcast_convert_type(
        (G >> 16).astype(jnp.int16), jnp.bfloat16).astype(jnp.float32)
    xg = xg_ref[...][:, :3]
    xq = xq_ref[0]
    xqr = jnp.broadcast_to(xq[:, None, :], (tq, k, 3)).reshape(tq * k, 3)
    rel = xqr - xg
    hd = jnp.maximum(
        jnp.dot(rel, d_w1_ref[...], preferred_element_type=jnp.float32)
        + d_b1_ref[...], 0.0)
    hdb = hd.astype(jnp.bfloat16)
    posg = jnp.dot(hdb, wdg_ref[...], preferred_element_type=jnp.float32)
    cvec = cvec_ref[0]
    hidden = jnp.maximum(cvec - Kg + posg, 0.0).astype(jnp.bfloat16)
    logits = (jnp.dot(hidden, g_w2_ref[...],
                      preferred_element_type=jnp.float32) + g_b2_ref[...])
    pos2 = (jnp.dot(hdb, d_w2_ref[...], preferred_element_type=jnp.float32)
            + d_b2_ref[...])
    w = V + pos2

    L3 = logits.reshape(tq, k, dim)
    W3 = w.reshape(tq, k, dim)
    lg = lgl_ref[0]
    m = jnp.maximum(jnp.max(L3, axis=1), lg)
    e = jnp.exp(L3 - m[:, None, :])
    eg = jnp.exp(lg - m)
    denom = jnp.sum(e, axis=1) + eg
    num = jnp.sum(e * W3, axis=1) + eg * vg_ref[0]
    out_ref[0] = num / denom


def _attend_call(g2, gx2, xyz_q, cvec, lgl, vg, wdg, d_w1, d_b1, d_w2, d_b2,
                 g_w2, g_b2):
    B, NQ, _ = xyz_q.shape
    DIM = wdg.shape[0]
    nt = NQ // TQ_ATT
    R = TQ_ATT * NNEIGH
    full = lambda s: pl.BlockSpec(s, lambda b, t: tuple(0 for _ in s))
    return pl.pallas_call(
        _attend_body,
        grid=(B, nt),
        in_specs=[
            pl.BlockSpec((R, TBLW), lambda b, t: (b * nt + t, 0)),
            pl.BlockSpec((R, XYZW), lambda b, t: (b * nt + t, 0)),
            pl.BlockSpec((1, TQ_ATT, 3), lambda b, t: (b, t, 0)),
            pl.BlockSpec((1, 1, DIM), lambda b, t: (b, 0, 0)),
            pl.BlockSpec((1, 1, DIM), lambda b, t: (b, 0, 0)),
            pl.BlockSpec((1, 1, DIM), lambda b, t: (b, 0, 0)),
            full((DIM, DIM)), full((3, DIM)), full((1, DIM)),
            full((DIM, DIM)), full((1, DIM)), full((DIM, DIM)),
            full((1, DIM)),
        ],
        out_specs=pl.BlockSpec((1, TQ_ATT, DIM), lambda b, t: (b, t, 0)),
        out_shape=jax.ShapeDtypeStruct((B, NQ, DIM), jnp.float32),
    )(g2, gx2, xyz_q, cvec, lgl, vg, wdg, d_w1, d_b1, d_w2, d_b2, g_w2, g_b2)


def kernel(xyz_q, lat_rep, xyz, points, W_qs, W_ks, W_vs, W_kg, W_vg,
           d_w1, d_b1, d_w2, d_b2, g_w1, g_b1, g_w2, g_b2):
    B, NQ, _ = xyz_q.shape
    N = xyz.shape[1]
    DIM = W_qs.shape[1]
    r2 = lambda v: v.reshape(1, DIM)

    table, cvec, lgl, vg, wdg, d_w2b, g_w2b = _prep_call(
        points, lat_rep, W_qs, W_ks, W_vs, W_kg, W_vg, r2(d_b2),
        g_w1, r2(g_b1), g_w2, r2(g_b2), d_w2)
    xyzT = xyz.transpose(0, 2, 1)
    xyzt = jnp.pad(xyz, ((0, 0), (0, 0), (0, XYZW - 3))).reshape(B * N, XYZW)
    r3 = lambda a: a.reshape(B, 1, DIM)
    cvec3, lgl3, vg3 = r3(cvec), r3(lgl), r3(vg)

    hb = B // 2
    outs = []
    gidx_h = [
        _topk_call(xyz_q[h * hb:(h + 1) * hb], xyzT[h * hb:(h + 1) * hb],
                   boff=h * hb).reshape(-1)
        for h in range(2)
    ]
    for h in range(2):
        s = slice(h * hb, (h + 1) * hb)
        g = _gather_call(table, gidx_h[h], TBLW, jnp.int32, True)
        gx = _gather_call(xyzt, gidx_h[h], XYZW, jnp.float32, False)
        outs.append(_attend_call(
            g, gx, xyz_q[s], cvec3[s], lgl3[s], vg3[s], wdg,
            d_w1, r2(d_b1), d_w2b, r2(d_b2), g_w2b, r2(g_b2)))
    return jnp.concatenate(outs, axis=0)
